# Initial kernel scaffold; baseline (speedup 1.0000x reference)
import jax
import jax.numpy as jnp
from jax.experimental import pallas as pl


def kernel(x, RWPE, adj_t, batch, index, bias, alpha, W_in, W_pe, W_self, W_agg, enc_W1, enc_b1, enc_W2, enc_b2, prototypes, dec_W1, dec_b1, dec_W2, dec_b2, alpha_p):
    raise NotImplementedError("write your pallas kernel here")


# speedup vs baseline: 3.6661x; 3.6661x over previous
import functools

import jax
import jax.numpy as jnp
import numpy as np
from jax import lax
from jax.experimental import pallas as pl
from jax.experimental.pallas import tpu as pltpu
from jax.experimental.pallas import tpu_sc as plsc

N = 10000
E = 320000
D = 128
PE = 16
G = 512
P = 64
H = 4
DH = D // H
K1 = 8
NC_OUT = 10
HID = 256

SC_CORES = 2
SC_SUBCORES = 16
NW = SC_CORES * SC_SUBCORES
EPW = E // NW
CB = 80
NCHUNK = EPW // CB
N_PAD = 10240
RPT = N_PAD // SC_SUBCORES
DEGW = 16

BN = 1000
NBLK = N // BN

def _dot(a, b):
    return jnp.dot(a, b, preferred_element_type=jnp.float32)


def _dotx(a, b):
    return jnp.dot(a, b, preferred_element_type=jnp.float32,
                   precision=lax.Precision.HIGHEST)


def _h0_body(x_ref, pe_ref, wi_ref, wp_ref, o_ref):
    o_ref[...] = _dot(x_ref[...], wi_ref[...]) + _dot(pe_ref[...], wp_ref[...])


def _compute_h0(x, RWPE, W_in, W_pe):
    return pl.pallas_call(
        _h0_body,
        grid=(NBLK,),
        in_specs=[
            pl.BlockSpec((BN, D), lambda i: (i, 0)),
            pl.BlockSpec((BN, PE), lambda i: (i, 0)),
            pl.BlockSpec((D, D), lambda i: (0, 0)),
            pl.BlockSpec((PE, D), lambda i: (0, 0)),
        ],
        out_specs=pl.BlockSpec((BN, D), lambda i: (i, 0)),
        out_shape=jax.ShapeDtypeStruct((N, D), jnp.float32),
    )(x, RWPE, W_in, W_pe)


def _sc_agg_body(h0_hbm, src_hbm, dst_hbm, zrow_hbm, agg_hbm,
                 idx_s, idx_d, rows, acc, sem):
    cid = lax.axis_index("c")
    sid = lax.axis_index("s")
    wid = sid * SC_CORES + cid
    r0 = sid * RPT
    pltpu.sync_copy(zrow_hbm, acc.at[pl.ds(r0, RPT)])
    plsc.subcore_barrier()
    ebase = wid * EPW

    def chunk(k, carry):
        b = ebase + k * CB
        pltpu.sync_copy(src_hbm.at[pl.ds(b, CB)], idx_s)
        pltpu.sync_copy(dst_hbm.at[pl.ds(b, CB)], idx_d)
        pltpu.async_copy(h0_hbm.at[idx_s], rows, sem).wait()
        pltpu.sync_copy(rows, acc.at[idx_d], add=True)
        return carry

    lax.fori_loop(0, NCHUNK, chunk, 0)
    plsc.subcore_barrier()
    pltpu.sync_copy(acc.at[pl.ds(r0, RPT)], agg_hbm.at[cid, pl.ds(r0, RPT)])


def _sc_deg_body(dst_hbm, zrow_hbm, ones_hbm, deg_hbm, idx_d, onesb, dacc, sem):
    cid = lax.axis_index("c")
    sid = lax.axis_index("s")
    wid = sid * SC_CORES + cid
    r0 = sid * RPT
    pltpu.sync_copy(zrow_hbm, dacc.at[pl.ds(r0, RPT)])
    pltpu.sync_copy(ones_hbm, onesb)
    plsc.subcore_barrier()
    ebase = wid * EPW

    def chunk(k, carry):
        b = ebase + k * CB
        pltpu.sync_copy(dst_hbm.at[pl.ds(b, CB)], idx_d)
        pltpu.sync_copy(onesb, dacc.at[idx_d], add=True)
        return carry

    lax.fori_loop(0, NCHUNK, chunk, 0)
    plsc.subcore_barrier()
    pltpu.sync_copy(dacc.at[pl.ds(r0, RPT)], deg_hbm.at[cid, pl.ds(r0, RPT)])


def _sc_mesh():
    return plsc.VectorSubcoreMesh(core_axis_name="c", subcore_axis_name="s",
                                  num_cores=SC_CORES, num_subcores=SC_SUBCORES)


@functools.cache
def _sc_agg_kernel():
    return functools.partial(
        pl.kernel,
        out_type=jax.ShapeDtypeStruct((SC_CORES, N_PAD, D), jnp.float32),
        mesh=_sc_mesh(),
        scratch_types=(
            pltpu.VMEM((CB,), jnp.int32),
            pltpu.VMEM((CB,), jnp.int32),
            pltpu.VMEM((CB, D), jnp.float32),
            pltpu.VMEM_SHARED((N_PAD, D), jnp.float32),
            pltpu.SemaphoreType.DMA,
        ),
    )(_sc_agg_body)


@functools.cache
def _sc_deg_kernel():
    return functools.partial(
        pl.kernel,
        out_type=jax.ShapeDtypeStruct((SC_CORES, N_PAD, D), jnp.float32),
        mesh=_sc_mesh(),
        scratch_types=(
            pltpu.VMEM((CB,), jnp.int32),
            pltpu.VMEM((CB, D), jnp.float32),
            pltpu.VMEM_SHARED((N_PAD, D), jnp.float32),
            pltpu.SemaphoreType.DMA,
        ),
    )(_sc_deg_body)


def _sc_edge_agg(h0, src, dst, zrow):
    return _sc_agg_kernel()(h0, src, dst, zrow)


def _sc_deg(dst, zrow, onesh):
    return _sc_deg_kernel()(dst, zrow, onesh)


def _pool_body(aggp_ref, degp_ref, h0_ref, wa_ref, ws_ref, b_ref,
               ps_ref, cnt_ref):
    i = pl.program_id(0)
    deg = jnp.maximum(degp_ref[0] + degp_ref[1], 1.0)
    agg = (aggp_ref[0] + aggp_ref[1]) / deg
    h = jnp.maximum(_dot(agg, wa_ref[...]) + _dot(h0_ref[...], ws_ref[...]), 0.0)
    seg = b_ref[...]
    gid = lax.broadcasted_iota(jnp.int32, (BN, G), 1).astype(jnp.float32)
    oh = (seg == gid).astype(jnp.float32)
    dn = (((0,), (0,)), ((), ()))
    hp = lax.Precision.HIGHEST
    psum = lax.dot_general(oh, h, dn,
                           preferred_element_type=jnp.float32, precision=hp)
    csum = lax.dot_general(oh, jnp.ones((BN, D), jnp.float32), dn,
                           preferred_element_type=jnp.float32, precision=hp)

    @pl.when(i == 0)
    def _():
        ps_ref[...] = jnp.zeros_like(ps_ref)
        cnt_ref[...] = jnp.zeros_like(cnt_ref)

    ps_ref[...] += psum
    cnt_ref[...] += csum


def _pool(aggp, degp, h0, W_agg, W_self, batchf):
    return pl.pallas_call(
        _pool_body,
        grid=(NBLK,),
        in_specs=[
            pl.BlockSpec((SC_CORES, BN, D), lambda i: (0, i, 0)),
            pl.BlockSpec((SC_CORES, BN, D), lambda i: (0, i, 0)),
            pl.BlockSpec((BN, D), lambda i: (i, 0)),
            pl.BlockSpec((D, D), lambda i: (0, 0)),
            pl.BlockSpec((D, D), lambda i: (0, 0)),
            pl.BlockSpec((BN, 1), lambda i: (i, 0)),
        ],
        out_specs=[
            pl.BlockSpec((G, D), lambda i: (0, 0)),
            pl.BlockSpec((G, D), lambda i: (0, 0)),
        ],
        out_shape=[
            jax.ShapeDtypeStruct((G, D), jnp.float32),
            jax.ShapeDtypeStruct((G, D), jnp.float32),
        ],
    )(aggp, degp, h0, W_agg, W_self, batchf)


def _route_body(ps_ref, cnt_ref, ew1, eb1, ew2, eb2, ptT, pt, br,
                dw1, db1, dw2, db2, ap, lp_ref, ti_ref):
    pooled = ps_ref[...] / jnp.maximum(cnt_ref[...], 1.0)
    t = jnp.maximum(_dot(pooled, ew1[...]) + eb1[...], 0.0)
    z = _dot(t, ew2[...]) + eb2[...]

    iota_p = lax.broadcasted_iota(jnp.int32, (G, P), 1)
    col32 = lax.broadcasted_iota(jnp.int32, (G, H * K1), 1)
    scale = np.float32(1.0 / np.sqrt(DH))
    ti = jnp.zeros((G, H * K1), jnp.int32)
    c_parts = []
    for hh in range(H):
        zh = z[:, hh * DH:(hh + 1) * DH]
        lg = _dot(zh, ptT[hh]) * scale
        work = lg + br[...]
        sels, ohs, idxs = [], [], []
        for j in range(K1):
            m = jnp.max(work, axis=1, keepdims=True)
            idx = jnp.min(jnp.where(work == m, iota_p, P), axis=1, keepdims=True)
            oh = iota_p == idx
            sel = jnp.sum(jnp.where(oh, lg, 0.0), axis=1, keepdims=True)
            work = jnp.where(oh, np.float32(-3.0e38), work)
            sels.append(sel)
            ohs.append(oh)
            idxs.append(idx)
        m8 = sels[0]
        for j in range(1, K1):
            m8 = jnp.maximum(m8, sels[j])
        es = [jnp.exp(s - m8) for s in sels]
        tot = es[0]
        for j in range(1, K1):
            tot = tot + es[j]
        rt = 1.0 / tot
        wcomb = jnp.zeros((G, P), jnp.float32)
        for j in range(K1):
            wcomb = wcomb + jnp.where(ohs[j], es[j] * rt, 0.0)
        c_parts.append(_dotx(wcomb, pt[hh]))
        for j in range(K1):
            ti = jnp.where(col32 == (hh * K1 + j), idxs[j], ti)

    c = jnp.concatenate(c_parts, axis=1)
    sg = 1.0 / (1.0 + jnp.exp(-ap[0, 0]))
    o = sg * c
    t2 = jnp.maximum(_dot(o, dw1[...]) + db1[...], 0.0)
    lgts = _dot(t2, dw2[...]) + db2[...]
    colD = lax.broadcasted_iota(jnp.int32, (G, 128), 1)
    valid = colD < NC_OUT
    mx = jnp.max(jnp.where(valid, lgts, np.float32(-3.0e38)), axis=1, keepdims=True)
    ex = jnp.where(valid, jnp.exp(lgts - mx), 0.0)
    lse = jnp.log(jnp.sum(ex, axis=1, keepdims=True))
    lp = lgts - mx - lse
    lp_ref[...] = lax.slice(lp, (0, 0), (G, NC_OUT))
    ti_ref[...] = ti


def _route(psum, cnt, enc_W1, eb1, enc_W2, eb2, protoT, protos, biasr,
           dec_W1, db1, dW2p, db2p, ap):
    return pl.pallas_call(
        _route_body,
        out_shape=[
            jax.ShapeDtypeStruct((G, NC_OUT), jnp.float32),
            jax.ShapeDtypeStruct((G, H * K1), jnp.int32),
        ],
    )(psum, cnt, enc_W1, eb1, enc_W2, eb2, protoT, protos, biasr,
      dec_W1, db1, dW2p, db2p, ap)


def kernel(x, RWPE, adj_t, batch, index, bias, alpha,
           W_in, W_pe, W_self, W_agg,
           enc_W1, enc_b1, enc_W2, enc_b2,
           prototypes, dec_W1, dec_b1, dec_W2, dec_b2, alpha_p):
    src = adj_t[0]
    dst = adj_t[1]

    h0 = _compute_h0(x, RWPE, W_in, W_pe)

    zrow = jnp.zeros((RPT, D), jnp.float32)
    onesh = jnp.ones((CB, D), jnp.float32)
    degp = _sc_deg(dst, zrow, onesh)
    aggp = _sc_edge_agg(h0, src, dst, zrow)

    batchf = batch.astype(jnp.float32).reshape(N, 1)
    psum, cnt = _pool(aggp, degp, h0, W_agg, W_self, batchf)

    protoT = jnp.transpose(prototypes, (0, 2, 1))
    lp, ti = _route(
        psum, cnt,
        enc_W1, enc_b1.reshape(1, HID), enc_W2, enc_b2.reshape(1, D),
        protoT, prototypes, bias.reshape(1, P),
        dec_W1, dec_b1.reshape(1, HID),
        jnp.pad(dec_W2, ((0, 0), (0, 128 - NC_OUT))),
        jnp.pad(dec_b2, (0, 128 - NC_OUT)).reshape(1, 128),
        jnp.asarray(alpha_p, jnp.float32).reshape(1, 1),
    )
    return (lp, ti.reshape(G, H, K1))


# speedup vs baseline: 4.0456x; 1.1035x over previous
import functools

import jax
import jax.numpy as jnp
import numpy as np
from jax import lax
from jax.experimental import pallas as pl
from jax.experimental.pallas import tpu as pltpu
from jax.experimental.pallas import tpu_sc as plsc

N = 10000
E = 320000
D = 128
PE = 16
G = 512
P = 64
H = 4
DH = D // H
K1 = 8
NC_OUT = 10
HID = 256

SC_CORES = 2
SC_SUBCORES = 16
NW = SC_CORES * SC_SUBCORES
EPW = E // NW
CB = 80
NCHUNK = EPW // CB
N_PAD = 10240
RPT = N_PAD // SC_SUBCORES
DEGW = 16

BN = 1000
NBLK = N // BN

def _dot(a, b):
    return jnp.dot(a, b, preferred_element_type=jnp.float32)


def _dotx(a, b):
    return jnp.dot(a, b, preferred_element_type=jnp.float32,
                   precision=lax.Precision.HIGHEST)


def _h0_body(x_ref, pe_ref, wi_ref, wp_ref, o_ref):
    o_ref[...] = _dot(x_ref[...], wi_ref[...]) + _dot(pe_ref[...], wp_ref[...])


def _compute_h0(x, RWPE, W_in, W_pe):
    return pl.pallas_call(
        _h0_body,
        grid=(NBLK,),
        in_specs=[
            pl.BlockSpec((BN, D), lambda i: (i, 0)),
            pl.BlockSpec((BN, PE), lambda i: (i, 0)),
            pl.BlockSpec((D, D), lambda i: (0, 0)),
            pl.BlockSpec((PE, D), lambda i: (0, 0)),
        ],
        out_specs=pl.BlockSpec((BN, D), lambda i: (i, 0)),
        out_shape=jax.ShapeDtypeStruct((N, D), jnp.float32),
    )(x, RWPE, W_in, W_pe)


def _sc_agg_body(h0_hbm, src_hbm, dst_hbm, zrow_hbm, agg_hbm,
                 idx_s0, idx_d0, rows0, idx_s1, idx_d1, rows1, acc,
                 sem0, sem1):
    cid = lax.axis_index("c")
    sid = lax.axis_index("s")
    wid = sid * SC_CORES + cid
    r0 = sid * RPT
    pltpu.sync_copy(zrow_hbm, acc.at[pl.ds(r0, RPT)])
    plsc.subcore_barrier()
    ebase = wid * EPW
    bufs = ((idx_s0, idx_d0, rows0, sem0), (idx_s1, idx_d1, rows1, sem1))

    def issue(k, buf):
        b = ebase + k * CB
        pltpu.sync_copy(src_hbm.at[pl.ds(b, CB)], buf[0])
        pltpu.sync_copy(dst_hbm.at[pl.ds(b, CB)], buf[1])
        pltpu.async_copy(h0_hbm.at[buf[0]], buf[2], buf[3])

    def drain_scatter(k, buf):
        pltpu.make_async_copy(h0_hbm.at[buf[0]], buf[2], buf[3]).wait()
        pltpu.sync_copy(buf[2], acc.at[buf[1]], add=True)

    issue(0, bufs[0])

    def pair(j, carry):
        k = 2 * j
        pltpu.make_async_copy(h0_hbm.at[bufs[0][0]], bufs[0][2], bufs[0][3]).wait()
        issue(k + 1, bufs[1])
        pltpu.sync_copy(bufs[0][2], acc.at[bufs[0][1]], add=True)
        pltpu.make_async_copy(h0_hbm.at[bufs[1][0]], bufs[1][2], bufs[1][3]).wait()
        issue(k + 2, bufs[0])
        pltpu.sync_copy(bufs[1][2], acc.at[bufs[1][1]], add=True)
        return carry

    lax.fori_loop(0, (NCHUNK - 1) // 2, pair, 0)
    drain_scatter(NCHUNK - 1, bufs[0])
    plsc.subcore_barrier()
    pltpu.sync_copy(acc.at[pl.ds(r0, RPT)], agg_hbm.at[cid, pl.ds(r0, RPT)])


def _sc_deg_body(dst_hbm, zrow_hbm, ones_hbm, deg_hbm, idx_d, onesb, dacc, sem):
    cid = lax.axis_index("c")
    sid = lax.axis_index("s")
    wid = sid * SC_CORES + cid
    r0 = sid * RPT
    pltpu.sync_copy(zrow_hbm, dacc.at[pl.ds(r0, RPT)])
    pltpu.sync_copy(ones_hbm, onesb)
    plsc.subcore_barrier()
    ebase = wid * EPW

    def chunk(k, carry):
        b = ebase + k * CB
        pltpu.sync_copy(dst_hbm.at[pl.ds(b, CB)], idx_d)
        pltpu.sync_copy(onesb, dacc.at[idx_d], add=True)
        return carry

    lax.fori_loop(0, NCHUNK, chunk, 0)
    plsc.subcore_barrier()
    pltpu.sync_copy(dacc.at[pl.ds(r0, RPT)], deg_hbm.at[cid, pl.ds(r0, RPT)])


def _sc_mesh():
    return plsc.VectorSubcoreMesh(core_axis_name="c", subcore_axis_name="s",
                                  num_cores=SC_CORES, num_subcores=SC_SUBCORES)


@functools.cache
def _sc_agg_kernel():
    return functools.partial(
        pl.kernel,
        out_type=jax.ShapeDtypeStruct((SC_CORES, N_PAD, D), jnp.float32),
        mesh=_sc_mesh(),
        scratch_types=(
            pltpu.VMEM((CB,), jnp.int32),
            pltpu.VMEM((CB,), jnp.int32),
            pltpu.VMEM((CB, D), jnp.float32),
            pltpu.VMEM((CB,), jnp.int32),
            pltpu.VMEM((CB,), jnp.int32),
            pltpu.VMEM((CB, D), jnp.float32),
            pltpu.VMEM_SHARED((N_PAD, D), jnp.float32),
            pltpu.SemaphoreType.DMA,
            pltpu.SemaphoreType.DMA,
        ),
    )(_sc_agg_body)


@functools.cache
def _sc_deg_kernel():
    return functools.partial(
        pl.kernel,
        out_type=jax.ShapeDtypeStruct((SC_CORES, N_PAD, D), jnp.float32),
        mesh=_sc_mesh(),
        scratch_types=(
            pltpu.VMEM((CB,), jnp.int32),
            pltpu.VMEM((CB, D), jnp.float32),
            pltpu.VMEM_SHARED((N_PAD, D), jnp.float32),
            pltpu.SemaphoreType.DMA,
        ),
    )(_sc_deg_body)


def _sc_edge_agg(h0, src, dst, zrow):
    return _sc_agg_kernel()(h0, src, dst, zrow)


def _sc_deg(dst, zrow, onesh):
    return _sc_deg_kernel()(dst, zrow, onesh)


def _pool_body(aggp_ref, degp_ref, h0_ref, wa_ref, ws_ref, b_ref,
               ps_ref, cnt_ref):
    i = pl.program_id(0)
    deg = jnp.maximum(degp_ref[0] + degp_ref[1], 1.0)
    agg = (aggp_ref[0] + aggp_ref[1]) / deg
    h = jnp.maximum(_dot(agg, wa_ref[...]) + _dot(h0_ref[...], ws_ref[...]), 0.0)
    seg = b_ref[...]
    gid = lax.broadcasted_iota(jnp.int32, (BN, G), 1).astype(jnp.float32)
    oh = (seg == gid).astype(jnp.float32)
    dn = (((0,), (0,)), ((), ()))
    hp = lax.Precision.HIGHEST
    psum = lax.dot_general(oh, h, dn,
                           preferred_element_type=jnp.float32, precision=hp)
    csum = lax.dot_general(oh, jnp.ones((BN, D), jnp.float32), dn,
                           preferred_element_type=jnp.float32, precision=hp)

    @pl.when(i == 0)
    def _():
        ps_ref[...] = jnp.zeros_like(ps_ref)
        cnt_ref[...] = jnp.zeros_like(cnt_ref)

    ps_ref[...] += psum
    cnt_ref[...] += csum


def _pool(aggp, degp, h0, W_agg, W_self, batchf):
    return pl.pallas_call(
        _pool_body,
        grid=(NBLK,),
        in_specs=[
            pl.BlockSpec((SC_CORES, BN, D), lambda i: (0, i, 0)),
            pl.BlockSpec((SC_CORES, BN, D), lambda i: (0, i, 0)),
            pl.BlockSpec((BN, D), lambda i: (i, 0)),
            pl.BlockSpec((D, D), lambda i: (0, 0)),
            pl.BlockSpec((D, D), lambda i: (0, 0)),
            pl.BlockSpec((BN, 1), lambda i: (i, 0)),
        ],
        out_specs=[
            pl.BlockSpec((G, D), lambda i: (0, 0)),
            pl.BlockSpec((G, D), lambda i: (0, 0)),
        ],
        out_shape=[
            jax.ShapeDtypeStruct((G, D), jnp.float32),
            jax.ShapeDtypeStruct((G, D), jnp.float32),
        ],
    )(aggp, degp, h0, W_agg, W_self, batchf)


def _route_body(ps_ref, cnt_ref, ew1, eb1, ew2, eb2, ptT, pt, br,
                dw1, db1, dw2, db2, ap, lp_ref, ti_ref):
    pooled = ps_ref[...] / jnp.maximum(cnt_ref[...], 1.0)
    t = jnp.maximum(_dot(pooled, ew1[...]) + eb1[...], 0.0)
    z = _dot(t, ew2[...]) + eb2[...]

    iota_p = lax.broadcasted_iota(jnp.int32, (G, P), 1)
    col32 = lax.broadcasted_iota(jnp.int32, (G, H * K1), 1)
    scale = np.float32(1.0 / np.sqrt(DH))
    ti = jnp.zeros((G, H * K1), jnp.int32)
    c_parts = []
    for hh in range(H):
        zh = z[:, hh * DH:(hh + 1) * DH]
        lg = _dot(zh, ptT[hh]) * scale
        work = lg + br[...]
        sels, ohs, idxs = [], [], []
        for j in range(K1):
            m = jnp.max(work, axis=1, keepdims=True)
            idx = jnp.min(jnp.where(work == m, iota_p, P), axis=1, keepdims=True)
            oh = iota_p == idx
            sel = jnp.sum(jnp.where(oh, lg, 0.0), axis=1, keepdims=True)
            work = jnp.where(oh, np.float32(-3.0e38), work)
            sels.append(sel)
            ohs.append(oh)
            idxs.append(idx)
        m8 = sels[0]
        for j in range(1, K1):
            m8 = jnp.maximum(m8, sels[j])
        es = [jnp.exp(s - m8) for s in sels]
        tot = es[0]
        for j in range(1, K1):
            tot = tot + es[j]
        rt = 1.0 / tot
        wcomb = jnp.zeros((G, P), jnp.float32)
        for j in range(K1):
            wcomb = wcomb + jnp.where(ohs[j], es[j] * rt, 0.0)
        c_parts.append(_dotx(wcomb, pt[hh]))
        for j in range(K1):
            ti = jnp.where(col32 == (hh * K1 + j), idxs[j], ti)

    c = jnp.concatenate(c_parts, axis=1)
    sg = 1.0 / (1.0 + jnp.exp(-ap[0, 0]))
    o = sg * c
    t2 = jnp.maximum(_dot(o, dw1[...]) + db1[...], 0.0)
    lgts = _dot(t2, dw2[...]) + db2[...]
    colD = lax.broadcasted_iota(jnp.int32, (G, 128), 1)
    valid = colD < NC_OUT
    mx = jnp.max(jnp.where(valid, lgts, np.float32(-3.0e38)), axis=1, keepdims=True)
    ex = jnp.where(valid, jnp.exp(lgts - mx), 0.0)
    lse = jnp.log(jnp.sum(ex, axis=1, keepdims=True))
    lp = lgts - mx - lse
    lp_ref[...] = lax.slice(lp, (0, 0), (G, NC_OUT))
    ti_ref[...] = ti


def _route(psum, cnt, enc_W1, eb1, enc_W2, eb2, protoT, protos, biasr,
           dec_W1, db1, dW2p, db2p, ap):
    return pl.pallas_call(
        _route_body,
        out_shape=[
            jax.ShapeDtypeStruct((G, NC_OUT), jnp.float32),
            jax.ShapeDtypeStruct((G, H * K1), jnp.int32),
        ],
    )(psum, cnt, enc_W1, eb1, enc_W2, eb2, protoT, protos, biasr,
      dec_W1, db1, dW2p, db2p, ap)


def kernel(x, RWPE, adj_t, batch, index, bias, alpha,
           W_in, W_pe, W_self, W_agg,
           enc_W1, enc_b1, enc_W2, enc_b2,
           prototypes, dec_W1, dec_b1, dec_W2, dec_b2, alpha_p):
    src = adj_t[0]
    dst = adj_t[1]

    h0 = _compute_h0(x, RWPE, W_in, W_pe)

    zrow = jnp.zeros((RPT, D), jnp.float32)
    onesh = jnp.ones((CB, D), jnp.float32)
    degp = _sc_deg(dst, zrow, onesh)
    aggp = _sc_edge_agg(h0, src, dst, zrow)

    batchf = batch.astype(jnp.float32).reshape(N, 1)
    psum, cnt = _pool(aggp, degp, h0, W_agg, W_self, batchf)

    protoT = jnp.transpose(prototypes, (0, 2, 1))
    lp, ti = _route(
        psum, cnt,
        enc_W1, enc_b1.reshape(1, HID), enc_W2, enc_b2.reshape(1, D),
        protoT, prototypes, bias.reshape(1, P),
        dec_W1, dec_b1.reshape(1, HID),
        jnp.pad(dec_W2, ((0, 0), (0, 128 - NC_OUT))),
        jnp.pad(dec_b2, (0, 128 - NC_OUT)).reshape(1, 128),
        jnp.asarray(alpha_p, jnp.float32).reshape(1, 1),
    )
    return (lp, ti.reshape(G, H, K1))


# speedup vs baseline: 5.0654x; 1.2521x over previous
import functools

import jax
import jax.numpy as jnp
import numpy as np
from jax import lax
from jax.experimental import pallas as pl
from jax.experimental.pallas import tpu as pltpu
from jax.experimental.pallas import tpu_sc as plsc

N = 10000
E = 320000
D = 128
PE = 16
G = 512
P = 64
H = 4
DH = D // H
K1 = 8
NC_OUT = 10
HID = 256

SC_CORES = 2
SC_SUBCORES = 16
NW = SC_CORES * SC_SUBCORES
EPW = E // NW
CB = 80
NCHUNK = EPW // CB
N_PAD = 10240
RPT = N_PAD // SC_SUBCORES
DEGW = 16

BN = 1000
NBLK = N // BN

def _dot(a, b):
    return jnp.dot(a, b, preferred_element_type=jnp.float32)


def _dotx(a, b):
    return jnp.dot(a, b, preferred_element_type=jnp.float32,
                   precision=lax.Precision.HIGHEST)


def _h0_body(x_ref, pe_ref, wi_ref, wp_ref, o_ref):
    o_ref[...] = _dot(x_ref[...], wi_ref[...]) + _dot(pe_ref[...], wp_ref[...])


def _compute_h0(x, RWPE, W_in, W_pe):
    return pl.pallas_call(
        _h0_body,
        grid=(NBLK,),
        in_specs=[
            pl.BlockSpec((BN, D), lambda i: (i, 0)),
            pl.BlockSpec((BN, PE), lambda i: (i, 0)),
            pl.BlockSpec((D, D), lambda i: (0, 0)),
            pl.BlockSpec((PE, D), lambda i: (0, 0)),
        ],
        out_specs=pl.BlockSpec((BN, D), lambda i: (i, 0)),
        out_shape=jax.ShapeDtypeStruct((N, D), jnp.float32),
    )(x, RWPE, W_in, W_pe)


def _sc_agg_body(h0_hbm, src_hbm, dst_hbm, zrow_hbm, agg_hbm,
                 sidx0, sidx1, didx, rows0, rows1, acc, sem0, sem1):
    cid = lax.axis_index("c")
    sid = lax.axis_index("s")
    wid = sid * SC_CORES + cid
    r0 = sid * RPT
    pltpu.sync_copy(dst_hbm.at[wid], didx)
    pltpu.sync_copy(zrow_hbm, acc.at[pl.ds(r0, RPT)])
    plsc.subcore_barrier()
    ebase = wid * EPW
    bufs = ((sidx0, rows0, sem0), (sidx1, rows1, sem1))

    def issue(k, buf):
        pltpu.sync_copy(src_hbm.at[pl.ds(ebase + k * CB, CB)], buf[0])
        pltpu.async_copy(h0_hbm.at[buf[0]], buf[1], buf[2])

    issue(0, bufs[0])

    def pair(j, carry):
        k = 2 * j
        pltpu.make_async_copy(h0_hbm.at[bufs[0][0]], bufs[0][1], bufs[0][2]).wait()
        issue(k + 1, bufs[1])
        pltpu.sync_copy(bufs[0][1], acc.at[didx.at[k]], add=True)
        pltpu.make_async_copy(h0_hbm.at[bufs[1][0]], bufs[1][1], bufs[1][2]).wait()
        issue(k + 2, bufs[0])
        pltpu.sync_copy(bufs[1][1], acc.at[didx.at[k + 1]], add=True)
        return carry

    lax.fori_loop(0, (NCHUNK - 1) // 2, pair, 0)
    pltpu.make_async_copy(h0_hbm.at[bufs[0][0]], bufs[0][1], bufs[0][2]).wait()
    pltpu.sync_copy(bufs[0][1], acc.at[didx.at[NCHUNK - 1]], add=True)
    plsc.subcore_barrier()
    pltpu.sync_copy(acc.at[pl.ds(r0, RPT)], agg_hbm.at[cid, pl.ds(r0, RPT)])


def _sc_deg_body(dst_hbm, zrow_hbm, ones_hbm, deg_hbm, didx, onesb, dacc, sem):
    cid = lax.axis_index("c")
    sid = lax.axis_index("s")
    wid = sid * SC_CORES + cid
    r0 = sid * RPT
    pltpu.sync_copy(dst_hbm.at[wid], didx)
    pltpu.sync_copy(zrow_hbm, dacc.at[pl.ds(r0, RPT)])
    pltpu.sync_copy(ones_hbm, onesb)
    plsc.subcore_barrier()

    def chunk(k, carry):
        pltpu.sync_copy(onesb, dacc.at[didx.at[k]], add=True)
        return carry

    lax.fori_loop(0, NCHUNK, chunk, 0)
    plsc.subcore_barrier()
    pltpu.sync_copy(dacc.at[pl.ds(r0, RPT)], deg_hbm.at[cid, pl.ds(r0, RPT)])


def _sc_mesh():
    return plsc.VectorSubcoreMesh(core_axis_name="c", subcore_axis_name="s",
                                  num_cores=SC_CORES, num_subcores=SC_SUBCORES)


@functools.cache
def _sc_agg_kernel():
    return functools.partial(
        pl.kernel,
        out_type=jax.ShapeDtypeStruct((SC_CORES, N_PAD, D), jnp.float32),
        mesh=_sc_mesh(),
        scratch_types=(
            pltpu.VMEM((CB,), jnp.int32),
            pltpu.VMEM((CB,), jnp.int32),
            pltpu.VMEM((NCHUNK, CB), jnp.int32),
            pltpu.VMEM((CB, D), jnp.float32),
            pltpu.VMEM((CB, D), jnp.float32),
            pltpu.VMEM_SHARED((N_PAD, D), jnp.float32),
            pltpu.SemaphoreType.DMA,
            pltpu.SemaphoreType.DMA,
        ),
    )(_sc_agg_body)


@functools.cache
def _sc_deg_kernel():
    return functools.partial(
        pl.kernel,
        out_type=jax.ShapeDtypeStruct((SC_CORES, N_PAD, D), jnp.float32),
        mesh=_sc_mesh(),
        scratch_types=(
            pltpu.VMEM((NCHUNK, CB), jnp.int32),
            pltpu.VMEM((CB, D), jnp.float32),
            pltpu.VMEM_SHARED((N_PAD, D), jnp.float32),
            pltpu.SemaphoreType.DMA,
        ),
    )(_sc_deg_body)


def _sc_edge_agg(h0, src, dst, zrow):
    return _sc_agg_kernel()(h0, src, dst.reshape(NW, NCHUNK, CB), zrow)


def _sc_deg(dst, zrow, onesh):
    return _sc_deg_kernel()(dst.reshape(NW, NCHUNK, CB), zrow, onesh)


def _pool_body(aggp_ref, degp_ref, h0_ref, wa_ref, ws_ref, b_ref,
               ps_ref, cnt_ref):
    i = pl.program_id(0)
    deg = jnp.maximum(degp_ref[0] + degp_ref[1], 1.0)
    agg = (aggp_ref[0] + aggp_ref[1]) / deg
    h = jnp.maximum(_dot(agg, wa_ref[...]) + _dot(h0_ref[...], ws_ref[...]), 0.0)
    seg = b_ref[...]
    gid = lax.broadcasted_iota(jnp.int32, (BN, G), 1).astype(jnp.float32)
    oh = (seg == gid).astype(jnp.float32)
    dn = (((0,), (0,)), ((), ()))
    hp = lax.Precision.HIGHEST
    psum = lax.dot_general(oh, h, dn,
                           preferred_element_type=jnp.float32, precision=hp)
    csum = lax.dot_general(oh, jnp.ones((BN, D), jnp.float32), dn,
                           preferred_element_type=jnp.float32, precision=hp)

    @pl.when(i == 0)
    def _():
        ps_ref[...] = jnp.zeros_like(ps_ref)
        cnt_ref[...] = jnp.zeros_like(cnt_ref)

    ps_ref[...] += psum
    cnt_ref[...] += csum


def _pool(aggp, degp, h0, W_agg, W_self, batchf):
    return pl.pallas_call(
        _pool_body,
        grid=(NBLK,),
        in_specs=[
            pl.BlockSpec((SC_CORES, BN, D), lambda i: (0, i, 0)),
            pl.BlockSpec((SC_CORES, BN, D), lambda i: (0, i, 0)),
            pl.BlockSpec((BN, D), lambda i: (i, 0)),
            pl.BlockSpec((D, D), lambda i: (0, 0)),
            pl.BlockSpec((D, D), lambda i: (0, 0)),
            pl.BlockSpec((BN, 1), lambda i: (i, 0)),
        ],
        out_specs=[
            pl.BlockSpec((G, D), lambda i: (0, 0)),
            pl.BlockSpec((G, D), lambda i: (0, 0)),
        ],
        out_shape=[
            jax.ShapeDtypeStruct((G, D), jnp.float32),
            jax.ShapeDtypeStruct((G, D), jnp.float32),
        ],
    )(aggp, degp, h0, W_agg, W_self, batchf)


def _route_body(ps_ref, cnt_ref, ew1, eb1, ew2, eb2, ptT, pt, br,
                dw1, db1, dw2, db2, ap, lp_ref, ti_ref):
    pooled = ps_ref[...] / jnp.maximum(cnt_ref[...], 1.0)
    t = jnp.maximum(_dot(pooled, ew1[...]) + eb1[...], 0.0)
    z = _dot(t, ew2[...]) + eb2[...]

    iota_p = lax.broadcasted_iota(jnp.int32, (G, P), 1)
    col32 = lax.broadcasted_iota(jnp.int32, (G, H * K1), 1)
    scale = np.float32(1.0 / np.sqrt(DH))
    ti = jnp.zeros((G, H * K1), jnp.int32)
    c_parts = []
    for hh in range(H):
        zh = z[:, hh * DH:(hh + 1) * DH]
        lg = _dot(zh, ptT[hh]) * scale
        work = lg + br[...]
        sels, ohs, idxs = [], [], []
        for j in range(K1):
            m = jnp.max(work, axis=1, keepdims=True)
            idx = jnp.min(jnp.where(work == m, iota_p, P), axis=1, keepdims=True)
            oh = iota_p == idx
            sel = jnp.sum(jnp.where(oh, lg, 0.0), axis=1, keepdims=True)
            work = jnp.where(oh, np.float32(-3.0e38), work)
            sels.append(sel)
            ohs.append(oh)
            idxs.append(idx)
        m8 = sels[0]
        for j in range(1, K1):
            m8 = jnp.maximum(m8, sels[j])
        es = [jnp.exp(s - m8) for s in sels]
        tot = es[0]
        for j in range(1, K1):
            tot = tot + es[j]
        rt = 1.0 / tot
        wcomb = jnp.zeros((G, P), jnp.float32)
        for j in range(K1):
            wcomb = wcomb + jnp.where(ohs[j], es[j] * rt, 0.0)
        c_parts.append(_dotx(wcomb, pt[hh]))
        for j in range(K1):
            ti = jnp.where(col32 == (hh * K1 + j), idxs[j], ti)

    c = jnp.concatenate(c_parts, axis=1)
    sg = 1.0 / (1.0 + jnp.exp(-ap[0, 0]))
    o = sg * c
    t2 = jnp.maximum(_dot(o, dw1[...]) + db1[...], 0.0)
    lgts = _dot(t2, dw2[...]) + db2[...]
    colD = lax.broadcasted_iota(jnp.int32, (G, 128), 1)
    valid = colD < NC_OUT
    mx = jnp.max(jnp.where(valid, lgts, np.float32(-3.0e38)), axis=1, keepdims=True)
    ex = jnp.where(valid, jnp.exp(lgts - mx), 0.0)
    lse = jnp.log(jnp.sum(ex, axis=1, keepdims=True))
    lp = lgts - mx - lse
    lp_ref[...] = lax.slice(lp, (0, 0), (G, NC_OUT))
    ti_ref[...] = ti


def _route(psum, cnt, enc_W1, eb1, enc_W2, eb2, protoT, protos, biasr,
           dec_W1, db1, dW2p, db2p, ap):
    return pl.pallas_call(
        _route_body,
        out_shape=[
            jax.ShapeDtypeStruct((G, NC_OUT), jnp.float32),
            jax.ShapeDtypeStruct((G, H * K1), jnp.int32),
        ],
    )(psum, cnt, enc_W1, eb1, enc_W2, eb2, protoT, protos, biasr,
      dec_W1, db1, dW2p, db2p, ap)


def kernel(x, RWPE, adj_t, batch, index, bias, alpha,
           W_in, W_pe, W_self, W_agg,
           enc_W1, enc_b1, enc_W2, enc_b2,
           prototypes, dec_W1, dec_b1, dec_W2, dec_b2, alpha_p):
    src = adj_t[0]
    dst = adj_t[1]

    h0 = _compute_h0(x, RWPE, W_in, W_pe)

    zrow = jnp.zeros((RPT, D), jnp.float32)
    onesh = jnp.ones((CB, D), jnp.float32)
    degp = _sc_deg(dst, zrow, onesh)
    aggp = _sc_edge_agg(h0, src, dst, zrow)

    batchf = batch.astype(jnp.float32).reshape(N, 1)
    psum, cnt = _pool(aggp, degp, h0, W_agg, W_self, batchf)

    protoT = jnp.transpose(prototypes, (0, 2, 1))
    lp, ti = _route(
        psum, cnt,
        enc_W1, enc_b1.reshape(1, HID), enc_W2, enc_b2.reshape(1, D),
        protoT, prototypes, bias.reshape(1, P),
        dec_W1, dec_b1.reshape(1, HID),
        jnp.pad(dec_W2, ((0, 0), (0, 128 - NC_OUT))),
        jnp.pad(dec_b2, (0, 128 - NC_OUT)).reshape(1, 128),
        jnp.asarray(alpha_p, jnp.float32).reshape(1, 1),
    )
    return (lp, ti.reshape(G, H, K1))


# speedup vs baseline: 5.8004x; 1.1451x over previous
import functools

import jax
import jax.numpy as jnp
import numpy as np
from jax import lax
from jax.experimental import pallas as pl
from jax.experimental.pallas import tpu as pltpu
from jax.experimental.pallas import tpu_sc as plsc

N = 10000
E = 320000
D = 128
PE = 16
G = 512
P = 64
H = 4
DH = D // H
K1 = 8
NC_OUT = 10
HID = 256

SC_CORES = 2
SC_SUBCORES = 16
NW = SC_CORES * SC_SUBCORES
EPW = E // NW
CB = 80
NCHUNK = EPW // CB
N_PAD = 10240
RPT = N_PAD // SC_SUBCORES
DEGW = 16

BN = 1000
NBLK = N // BN

def _dot(a, b):
    return jnp.dot(a, b, preferred_element_type=jnp.float32)


def _dotx(a, b):
    return jnp.dot(a, b, preferred_element_type=jnp.float32,
                   precision=lax.Precision.HIGHEST)


def _h0_body(x_ref, pe_ref, wi_ref, wp_ref, o_ref):
    o_ref[...] = _dot(x_ref[...], wi_ref[...]) + _dot(pe_ref[...], wp_ref[...])


def _compute_h0(x, RWPE, W_in, W_pe):
    return pl.pallas_call(
        _h0_body,
        grid=(NBLK,),
        in_specs=[
            pl.BlockSpec((BN, D), lambda i: (i, 0)),
            pl.BlockSpec((BN, PE), lambda i: (i, 0)),
            pl.BlockSpec((D, D), lambda i: (0, 0)),
            pl.BlockSpec((PE, D), lambda i: (0, 0)),
        ],
        out_specs=pl.BlockSpec((BN, D), lambda i: (i, 0)),
        out_shape=jax.ShapeDtypeStruct((N, D), jnp.float32),
    )(x, RWPE, W_in, W_pe)


def _sc_agg_body(h0_hbm, src_hbm, dst_hbm, zrow_hbm, agg_hbm,
                 sidx0, sidx1, didx, rows0, rows1, acc,
                 sem0, sem1, ssem0, ssem1):
    cid = lax.axis_index("c")
    sid = lax.axis_index("s")
    wid = sid * SC_CORES + cid
    r0 = sid * RPT
    pltpu.sync_copy(dst_hbm.at[wid], didx)
    pltpu.sync_copy(zrow_hbm, acc.at[pl.ds(r0, RPT)])
    plsc.subcore_barrier()
    ebase = wid * EPW
    bufs = ((sidx0, rows0, sem0, ssem0), (sidx1, rows1, sem1, ssem1))

    def src_slice(k):
        kk = jnp.minimum(k, NCHUNK - 1)
        return src_hbm.at[pl.ds(ebase + kk * CB, CB)]

    pltpu.sync_copy(src_slice(0), sidx0)
    pltpu.async_copy(h0_hbm.at[sidx0], rows0, sem0)
    pltpu.async_copy(src_slice(1), sidx1, ssem1)

    def step(k, cur, nxt):
        pltpu.make_async_copy(h0_hbm.at[cur[0]], cur[1], cur[2]).wait()
        pltpu.async_copy(src_slice(k + 2), cur[0], cur[3])
        pltpu.make_async_copy(src_slice(k + 1), nxt[0], nxt[3]).wait()
        pltpu.async_copy(h0_hbm.at[nxt[0]], nxt[1], nxt[2])
        pltpu.sync_copy(cur[1], acc.at[didx.at[k]], add=True)

    def pair(j, carry):
        k = 2 * j
        step(k, bufs[0], bufs[1])
        step(k + 1, bufs[1], bufs[0])
        return carry

    lax.fori_loop(0, (NCHUNK - 1) // 2, pair, 0)
    pltpu.make_async_copy(h0_hbm.at[bufs[0][0]], bufs[0][1], bufs[0][2]).wait()
    pltpu.sync_copy(bufs[0][1], acc.at[didx.at[NCHUNK - 1]], add=True)
    pltpu.make_async_copy(src_slice(NCHUNK - 1), bufs[1][0], bufs[1][3]).wait()
    plsc.subcore_barrier()
    pltpu.sync_copy(acc.at[pl.ds(r0, RPT)], agg_hbm.at[cid, pl.ds(r0, RPT)])


def _sc_deg_body(dst_hbm, zrow_hbm, ones_hbm, deg_hbm, didx, onesb, dacc, sem):
    cid = lax.axis_index("c")
    sid = lax.axis_index("s")
    wid = sid * SC_CORES + cid
    r0 = sid * RPT
    pltpu.sync_copy(dst_hbm.at[wid], didx)
    pltpu.sync_copy(zrow_hbm, dacc.at[pl.ds(r0, RPT)])
    pltpu.sync_copy(ones_hbm, onesb)
    plsc.subcore_barrier()

    def chunk(k, carry):
        pltpu.sync_copy(onesb, dacc.at[didx.at[k]], add=True)
        return carry

    lax.fori_loop(0, NCHUNK, chunk, 0)
    plsc.subcore_barrier()
    pltpu.sync_copy(dacc.at[pl.ds(r0, RPT)], deg_hbm.at[cid, pl.ds(r0, RPT)])


def _sc_mesh():
    return plsc.VectorSubcoreMesh(core_axis_name="c", subcore_axis_name="s",
                                  num_cores=SC_CORES, num_subcores=SC_SUBCORES)


@functools.cache
def _sc_agg_kernel():
    return functools.partial(
        pl.kernel,
        out_type=jax.ShapeDtypeStruct((SC_CORES, N_PAD, D), jnp.float32),
        mesh=_sc_mesh(),
        scratch_types=(
            pltpu.VMEM((CB,), jnp.int32),
            pltpu.VMEM((CB,), jnp.int32),
            pltpu.VMEM((NCHUNK, CB), jnp.int32),
            pltpu.VMEM((CB, D), jnp.float32),
            pltpu.VMEM((CB, D), jnp.float32),
            pltpu.VMEM_SHARED((N_PAD, D), jnp.float32),
            pltpu.SemaphoreType.DMA,
            pltpu.SemaphoreType.DMA,
            pltpu.SemaphoreType.DMA,
            pltpu.SemaphoreType.DMA,
        ),
    )(_sc_agg_body)


@functools.cache
def _sc_deg_kernel():
    return functools.partial(
        pl.kernel,
        out_type=jax.ShapeDtypeStruct((SC_CORES, N_PAD, D), jnp.float32),
        mesh=_sc_mesh(),
        scratch_types=(
            pltpu.VMEM((NCHUNK, CB), jnp.int32),
            pltpu.VMEM((CB, D), jnp.float32),
            pltpu.VMEM_SHARED((N_PAD, D), jnp.float32),
            pltpu.SemaphoreType.DMA,
        ),
    )(_sc_deg_body)


def _sc_edge_agg(h0, src, dst, zrow):
    return _sc_agg_kernel()(h0, src, dst.reshape(NW, NCHUNK, CB), zrow)


def _sc_deg(dst, zrow, onesh):
    return _sc_deg_kernel()(dst.reshape(NW, NCHUNK, CB), zrow, onesh)


def _pool_body(aggp_ref, degp_ref, h0_ref, wa_ref, ws_ref, b_ref,
               ps_ref, cnt_ref):
    i = pl.program_id(0)
    deg = jnp.maximum(degp_ref[0] + degp_ref[1], 1.0)
    agg = (aggp_ref[0] + aggp_ref[1]) / deg
    h = jnp.maximum(_dot(agg, wa_ref[...]) + _dot(h0_ref[...], ws_ref[...]), 0.0)
    seg = b_ref[...]
    gid = lax.broadcasted_iota(jnp.int32, (BN, G), 1).astype(jnp.float32)
    oh = (seg == gid).astype(jnp.float32)
    dn = (((0,), (0,)), ((), ()))
    hp = lax.Precision.HIGHEST
    psum = lax.dot_general(oh, h, dn,
                           preferred_element_type=jnp.float32, precision=hp)
    csum = lax.dot_general(oh, jnp.ones((BN, D), jnp.float32), dn,
                           preferred_element_type=jnp.float32, precision=hp)

    @pl.when(i == 0)
    def _():
        ps_ref[...] = jnp.zeros_like(ps_ref)
        cnt_ref[...] = jnp.zeros_like(cnt_ref)

    ps_ref[...] += psum
    cnt_ref[...] += csum


def _pool(aggp, degp, h0, W_agg, W_self, batchf):
    return pl.pallas_call(
        _pool_body,
        grid=(NBLK,),
        in_specs=[
            pl.BlockSpec((SC_CORES, BN, D), lambda i: (0, i, 0)),
            pl.BlockSpec((SC_CORES, BN, D), lambda i: (0, i, 0)),
            pl.BlockSpec((BN, D), lambda i: (i, 0)),
            pl.BlockSpec((D, D), lambda i: (0, 0)),
            pl.BlockSpec((D, D), lambda i: (0, 0)),
            pl.BlockSpec((BN, 1), lambda i: (i, 0)),
        ],
        out_specs=[
            pl.BlockSpec((G, D), lambda i: (0, 0)),
            pl.BlockSpec((G, D), lambda i: (0, 0)),
        ],
        out_shape=[
            jax.ShapeDtypeStruct((G, D), jnp.float32),
            jax.ShapeDtypeStruct((G, D), jnp.float32),
        ],
    )(aggp, degp, h0, W_agg, W_self, batchf)


def _route_body(ps_ref, cnt_ref, ew1, eb1, ew2, eb2, ptT, pt, br,
                dw1, db1, dw2, db2, ap, lp_ref, ti_ref):
    pooled = ps_ref[...] / jnp.maximum(cnt_ref[...], 1.0)
    t = jnp.maximum(_dot(pooled, ew1[...]) + eb1[...], 0.0)
    z = _dot(t, ew2[...]) + eb2[...]

    iota_p = lax.broadcasted_iota(jnp.int32, (G, P), 1)
    col32 = lax.broadcasted_iota(jnp.int32, (G, H * K1), 1)
    scale = np.float32(1.0 / np.sqrt(DH))
    ti = jnp.zeros((G, H * K1), jnp.int32)
    c_parts = []
    for hh in range(H):
        zh = z[:, hh * DH:(hh + 1) * DH]
        lg = _dot(zh, ptT[hh]) * scale
        work = lg + br[...]
        sels, ohs, idxs = [], [], []
        for j in range(K1):
            m = jnp.max(work, axis=1, keepdims=True)
            idx = jnp.min(jnp.where(work == m, iota_p, P), axis=1, keepdims=True)
            oh = iota_p == idx
            sel = jnp.sum(jnp.where(oh, lg, 0.0), axis=1, keepdims=True)
            work = jnp.where(oh, np.float32(-3.0e38), work)
            sels.append(sel)
            ohs.append(oh)
            idxs.append(idx)
        m8 = sels[0]
        for j in range(1, K1):
            m8 = jnp.maximum(m8, sels[j])
        es = [jnp.exp(s - m8) for s in sels]
        tot = es[0]
        for j in range(1, K1):
            tot = tot + es[j]
        rt = 1.0 / tot
        wcomb = jnp.zeros((G, P), jnp.float32)
        for j in range(K1):
            wcomb = wcomb + jnp.where(ohs[j], es[j] * rt, 0.0)
        c_parts.append(_dotx(wcomb, pt[hh]))
        for j in range(K1):
            ti = jnp.where(col32 == (hh * K1 + j), idxs[j], ti)

    c = jnp.concatenate(c_parts, axis=1)
    sg = 1.0 / (1.0 + jnp.exp(-ap[0, 0]))
    o = sg * c
    t2 = jnp.maximum(_dot(o, dw1[...]) + db1[...], 0.0)
    lgts = _dot(t2, dw2[...]) + db2[...]
    colD = lax.broadcasted_iota(jnp.int32, (G, 128), 1)
    valid = colD < NC_OUT
    mx = jnp.max(jnp.where(valid, lgts, np.float32(-3.0e38)), axis=1, keepdims=True)
    ex = jnp.where(valid, jnp.exp(lgts - mx), 0.0)
    lse = jnp.log(jnp.sum(ex, axis=1, keepdims=True))
    lp = lgts - mx - lse
    lp_ref[...] = lax.slice(lp, (0, 0), (G, NC_OUT))
    ti_ref[...] = ti


def _route(psum, cnt, enc_W1, eb1, enc_W2, eb2, protoT, protos, biasr,
           dec_W1, db1, dW2p, db2p, ap):
    return pl.pallas_call(
        _route_body,
        out_shape=[
            jax.ShapeDtypeStruct((G, NC_OUT), jnp.float32),
            jax.ShapeDtypeStruct((G, H * K1), jnp.int32),
        ],
    )(psum, cnt, enc_W1, eb1, enc_W2, eb2, protoT, protos, biasr,
      dec_W1, db1, dW2p, db2p, ap)


def kernel(x, RWPE, adj_t, batch, index, bias, alpha,
           W_in, W_pe, W_self, W_agg,
           enc_W1, enc_b1, enc_W2, enc_b2,
           prototypes, dec_W1, dec_b1, dec_W2, dec_b2, alpha_p):
    src = adj_t[0]
    dst = adj_t[1]

    h0 = _compute_h0(x, RWPE, W_in, W_pe)

    zrow = jnp.zeros((RPT, D), jnp.float32)
    onesh = jnp.ones((CB, D), jnp.float32)
    degp = _sc_deg(dst, zrow, onesh)
    aggp = _sc_edge_agg(h0, src, dst, zrow)

    batchf = batch.astype(jnp.float32).reshape(N, 1)
    psum, cnt = _pool(aggp, degp, h0, W_agg, W_self, batchf)

    protoT = jnp.transpose(prototypes, (0, 2, 1))
    lp, ti = _route(
        psum, cnt,
        enc_W1, enc_b1.reshape(1, HID), enc_W2, enc_b2.reshape(1, D),
        protoT, prototypes, bias.reshape(1, P),
        dec_W1, dec_b1.reshape(1, HID),
        jnp.pad(dec_W2, ((0, 0), (0, 128 - NC_OUT))),
        jnp.pad(dec_b2, (0, 128 - NC_OUT)).reshape(1, 128),
        jnp.asarray(alpha_p, jnp.float32).reshape(1, 1),
    )
    return (lp, ti.reshape(G, H, K1))


# speedup vs baseline: 5.9073x; 1.0184x over previous
import functools

import jax
import jax.numpy as jnp
import numpy as np
from jax import lax
from jax.experimental import pallas as pl
from jax.experimental.pallas import tpu as pltpu
from jax.experimental.pallas import tpu_sc as plsc

N = 10000
E = 320000
D = 128
PE = 16
G = 512
P = 64
H = 4
DH = D // H
K1 = 8
NC_OUT = 10
HID = 256

SC_CORES = 2
SC_SUBCORES = 16
NW = SC_CORES * SC_SUBCORES
EPW = E // NW
CB = 80
NCHUNK = EPW // CB
N_PAD = 10240
RPT = N_PAD // SC_SUBCORES
DEGW = 16

BN = 1000
NBLK = N // BN

def _dot(a, b):
    return jnp.dot(a, b, preferred_element_type=jnp.float32)


def _dotx(a, b):
    return jnp.dot(a, b, preferred_element_type=jnp.float32,
                   precision=lax.Precision.HIGHEST)


def _h0_body(x_ref, pe_ref, wi_ref, wp_ref, o_ref):
    o_ref[...] = _dot(x_ref[...], wi_ref[...]) + _dot(pe_ref[...], wp_ref[...])


def _compute_h0(x, RWPE, W_in, W_pe):
    return pl.pallas_call(
        _h0_body,
        grid=(NBLK,),
        in_specs=[
            pl.BlockSpec((BN, D), lambda i: (i, 0)),
            pl.BlockSpec((BN, PE), lambda i: (i, 0)),
            pl.BlockSpec((D, D), lambda i: (0, 0)),
            pl.BlockSpec((PE, D), lambda i: (0, 0)),
        ],
        out_specs=pl.BlockSpec((BN, D), lambda i: (i, 0)),
        out_shape=jax.ShapeDtypeStruct((N, D), jnp.float32),
    )(x, RWPE, W_in, W_pe)


def _sc_body(h0_hbm, src_hbm, dst_hbm, zrow_hbm, ones_hbm, agg_hbm, deg_hbm,
             sidx0, sidx1, didx, rows0, rows1, onesb, acc,
             sem0, sem1, ssem0, ssem1):
    cid = lax.axis_index("c")
    sid = lax.axis_index("s")
    wid = sid * SC_CORES + cid
    r0 = sid * RPT
    pltpu.sync_copy(dst_hbm.at[wid], didx)
    pltpu.sync_copy(ones_hbm, onesb)
    pltpu.sync_copy(zrow_hbm, acc.at[pl.ds(r0, RPT)])
    plsc.subcore_barrier()

    def dchunk(k, carry):
        pltpu.sync_copy(onesb, acc.at[didx.at[k]], add=True)
        return carry

    lax.fori_loop(0, NCHUNK, dchunk, 0)
    plsc.subcore_barrier()
    pltpu.sync_copy(acc.at[pl.ds(r0, RPT)], deg_hbm.at[cid, pl.ds(r0, RPT)])
    pltpu.sync_copy(zrow_hbm, acc.at[pl.ds(r0, RPT)])
    plsc.subcore_barrier()
    ebase = wid * EPW
    bufs = ((sidx0, rows0, sem0, ssem0), (sidx1, rows1, sem1, ssem1))

    def src_slice(k):
        kk = jnp.minimum(k, NCHUNK - 1)
        return src_hbm.at[pl.ds(ebase + kk * CB, CB)]

    pltpu.sync_copy(src_slice(0), sidx0)
    pltpu.async_copy(h0_hbm.at[sidx0], rows0, sem0)
    pltpu.async_copy(src_slice(1), sidx1, ssem1)

    def step(k, cur, nxt):
        pltpu.make_async_copy(h0_hbm.at[cur[0]], cur[1], cur[2]).wait()
        pltpu.async_copy(src_slice(k + 2), cur[0], cur[3])
        pltpu.make_async_copy(src_slice(k + 1), nxt[0], nxt[3]).wait()
        pltpu.async_copy(h0_hbm.at[nxt[0]], nxt[1], nxt[2])
        pltpu.sync_copy(cur[1], acc.at[didx.at[k]], add=True)

    def pair(j, carry):
        k = 2 * j
        step(k, bufs[0], bufs[1])
        step(k + 1, bufs[1], bufs[0])
        return carry

    lax.fori_loop(0, (NCHUNK - 1) // 2, pair, 0)
    pltpu.make_async_copy(h0_hbm.at[bufs[0][0]], bufs[0][1], bufs[0][2]).wait()
    pltpu.sync_copy(bufs[0][1], acc.at[didx.at[NCHUNK - 1]], add=True)
    pltpu.make_async_copy(src_slice(NCHUNK - 1), bufs[1][0], bufs[1][3]).wait()
    plsc.subcore_barrier()
    pltpu.sync_copy(acc.at[pl.ds(r0, RPT)], agg_hbm.at[cid, pl.ds(r0, RPT)])


def _sc_mesh():
    return plsc.VectorSubcoreMesh(core_axis_name="c", subcore_axis_name="s",
                                  num_cores=SC_CORES, num_subcores=SC_SUBCORES)


@functools.cache
def _sc_kernel():
    return functools.partial(
        pl.kernel,
        out_type=(
            jax.ShapeDtypeStruct((SC_CORES, N_PAD, D), jnp.float32),
            jax.ShapeDtypeStruct((SC_CORES, N_PAD, D), jnp.float32),
        ),
        mesh=_sc_mesh(),
        scratch_types=(
            pltpu.VMEM((CB,), jnp.int32),
            pltpu.VMEM((CB,), jnp.int32),
            pltpu.VMEM((NCHUNK, CB), jnp.int32),
            pltpu.VMEM((CB, D), jnp.float32),
            pltpu.VMEM((CB, D), jnp.float32),
            pltpu.VMEM((CB, D), jnp.float32),
            pltpu.VMEM_SHARED((N_PAD, D), jnp.float32),
            pltpu.SemaphoreType.DMA,
            pltpu.SemaphoreType.DMA,
            pltpu.SemaphoreType.DMA,
            pltpu.SemaphoreType.DMA,
        ),
    )(_sc_body)


def _sc_agg_deg(h0, src, dst, zrow, onesh):
    return _sc_kernel()(h0, src, dst.reshape(NW, NCHUNK, CB), zrow, onesh)


def _pool_body(aggp_ref, degp_ref, h0_ref, wa_ref, ws_ref, b_ref,
               ps_ref, cnt_ref):
    i = pl.program_id(0)
    deg = jnp.maximum(degp_ref[0] + degp_ref[1], 1.0)
    agg = (aggp_ref[0] + aggp_ref[1]) / deg
    h = jnp.maximum(_dot(agg, wa_ref[...]) + _dot(h0_ref[...], ws_ref[...]), 0.0)
    seg = b_ref[...]
    gid = lax.broadcasted_iota(jnp.int32, (BN, G), 1).astype(jnp.float32)
    oh = (seg == gid).astype(jnp.float32)
    dn = (((0,), (0,)), ((), ()))
    hp = lax.Precision.HIGHEST
    psum = lax.dot_general(oh, h, dn,
                           preferred_element_type=jnp.float32, precision=hp)
    csum = lax.dot_general(oh, jnp.ones((BN, D), jnp.float32), dn,
                           preferred_element_type=jnp.float32, precision=hp)

    @pl.when(i == 0)
    def _():
        ps_ref[...] = jnp.zeros_like(ps_ref)
        cnt_ref[...] = jnp.zeros_like(cnt_ref)

    ps_ref[...] += psum
    cnt_ref[...] += csum


def _pool(aggp, degp, h0, W_agg, W_self, batchf):
    return pl.pallas_call(
        _pool_body,
        grid=(NBLK,),
        in_specs=[
            pl.BlockSpec((SC_CORES, BN, D), lambda i: (0, i, 0)),
            pl.BlockSpec((SC_CORES, BN, D), lambda i: (0, i, 0)),
            pl.BlockSpec((BN, D), lambda i: (i, 0)),
            pl.BlockSpec((D, D), lambda i: (0, 0)),
            pl.BlockSpec((D, D), lambda i: (0, 0)),
            pl.BlockSpec((BN, 1), lambda i: (i, 0)),
        ],
        out_specs=[
            pl.BlockSpec((G, D), lambda i: (0, 0)),
            pl.BlockSpec((G, D), lambda i: (0, 0)),
        ],
        out_shape=[
            jax.ShapeDtypeStruct((G, D), jnp.float32),
            jax.ShapeDtypeStruct((G, D), jnp.float32),
        ],
    )(aggp, degp, h0, W_agg, W_self, batchf)


def _route_body(ps_ref, cnt_ref, ew1, eb1, ew2, eb2, ptT, pt, br,
                dw1, db1, dw2, db2, ap, lp_ref, ti_ref):
    pooled = ps_ref[...] / jnp.maximum(cnt_ref[...], 1.0)
    t = jnp.maximum(_dot(pooled, ew1[...]) + eb1[...], 0.0)
    z = _dot(t, ew2[...]) + eb2[...]

    iota_p = lax.broadcasted_iota(jnp.int32, (G, P), 1)
    col32 = lax.broadcasted_iota(jnp.int32, (G, H * K1), 1)
    scale = np.float32(1.0 / np.sqrt(DH))
    ti = jnp.zeros((G, H * K1), jnp.int32)
    c_parts = []
    for hh in range(H):
        zh = z[:, hh * DH:(hh + 1) * DH]
        lg = _dot(zh, ptT[hh]) * scale
        work = lg + br[...]
        sels, ohs, idxs = [], [], []
        for j in range(K1):
            m = jnp.max(work, axis=1, keepdims=True)
            idx = jnp.min(jnp.where(work == m, iota_p, P), axis=1, keepdims=True)
            oh = iota_p == idx
            sel = jnp.sum(jnp.where(oh, lg, 0.0), axis=1, keepdims=True)
            work = jnp.where(oh, np.float32(-3.0e38), work)
            sels.append(sel)
            ohs.append(oh)
            idxs.append(idx)
        m8 = sels[0]
        for j in range(1, K1):
            m8 = jnp.maximum(m8, sels[j])
        es = [jnp.exp(s - m8) for s in sels]
        tot = es[0]
        for j in range(1, K1):
            tot = tot + es[j]
        rt = 1.0 / tot
        wcomb = jnp.zeros((G, P), jnp.float32)
        for j in range(K1):
            wcomb = wcomb + jnp.where(ohs[j], es[j] * rt, 0.0)
        c_parts.append(_dotx(wcomb, pt[hh]))
        for j in range(K1):
            ti = jnp.where(col32 == (hh * K1 + j), idxs[j], ti)

    c = jnp.concatenate(c_parts, axis=1)
    sg = 1.0 / (1.0 + jnp.exp(-ap[0, 0]))
    o = sg * c
    t2 = jnp.maximum(_dot(o, dw1[...]) + db1[...], 0.0)
    lgts = _dot(t2, dw2[...]) + db2[...]
    colD = lax.broadcasted_iota(jnp.int32, (G, 128), 1)
    valid = colD < NC_OUT
    mx = jnp.max(jnp.where(valid, lgts, np.float32(-3.0e38)), axis=1, keepdims=True)
    ex = jnp.where(valid, jnp.exp(lgts - mx), 0.0)
    lse = jnp.log(jnp.sum(ex, axis=1, keepdims=True))
    lp = lgts - mx - lse
    lp_ref[...] = lax.slice(lp, (0, 0), (G, NC_OUT))
    ti_ref[...] = ti


def _route(psum, cnt, enc_W1, eb1, enc_W2, eb2, protoT, protos, biasr,
           dec_W1, db1, dW2p, db2p, ap):
    return pl.pallas_call(
        _route_body,
        out_shape=[
            jax.ShapeDtypeStruct((G, NC_OUT), jnp.float32),
            jax.ShapeDtypeStruct((G, H * K1), jnp.int32),
        ],
    )(psum, cnt, enc_W1, eb1, enc_W2, eb2, protoT, protos, biasr,
      dec_W1, db1, dW2p, db2p, ap)


def kernel(x, RWPE, adj_t, batch, index, bias, alpha,
           W_in, W_pe, W_self, W_agg,
           enc_W1, enc_b1, enc_W2, enc_b2,
           prototypes, dec_W1, dec_b1, dec_W2, dec_b2, alpha_p):
    src = adj_t[0]
    dst = adj_t[1]

    h0 = _compute_h0(x, RWPE, W_in, W_pe)

    zrow = jnp.zeros((RPT, D), jnp.float32)
    onesh = jnp.ones((CB, D), jnp.float32)
    aggp, degp = _sc_agg_deg(h0, src, dst, zrow, onesh)

    batchf = batch.astype(jnp.float32).reshape(N, 1)
    psum, cnt = _pool(aggp, degp, h0, W_agg, W_self, batchf)

    protoT = jnp.transpose(prototypes, (0, 2, 1))
    lp, ti = _route(
        psum, cnt,
        enc_W1, enc_b1.reshape(1, HID), enc_W2, enc_b2.reshape(1, D),
        protoT, prototypes, bias.reshape(1, P),
        dec_W1, dec_b1.reshape(1, HID),
        jnp.pad(dec_W2, ((0, 0), (0, 128 - NC_OUT))),
        jnp.pad(dec_b2, (0, 128 - NC_OUT)).reshape(1, 128),
        jnp.asarray(alpha_p, jnp.float32).reshape(1, 1),
    )
    return (lp, ti.reshape(G, H, K1))


# speedup vs baseline: 5.9483x; 1.0069x over previous
import functools

import jax
import jax.numpy as jnp
import numpy as np
from jax import lax
from jax.experimental import pallas as pl
from jax.experimental.pallas import tpu as pltpu
from jax.experimental.pallas import tpu_sc as plsc

N = 10000
E = 320000
D = 128
PE = 16
G = 512
P = 64
H = 4
DH = D // H
K1 = 8
NC_OUT = 10
HID = 256

SC_CORES = 2
SC_SUBCORES = 16
NW = SC_CORES * SC_SUBCORES
EPW = E // NW
CB = 80
NCHUNK = EPW // CB
N_PAD = 10240
RPT = N_PAD // SC_SUBCORES
DEGW = 16

BN = 1000
NBLK = N // BN

def _dot(a, b):
    return jnp.dot(a, b, preferred_element_type=jnp.float32)


def _dotx(a, b):
    return jnp.dot(a, b, preferred_element_type=jnp.float32,
                   precision=lax.Precision.HIGHEST)


def _h0_body(x_ref, pe_ref, wi_ref, wp_ref, o_ref):
    o_ref[...] = _dot(x_ref[...], wi_ref[...]) + _dot(pe_ref[...], wp_ref[...])


def _compute_h0(x, RWPE, W_in, W_pe):
    return pl.pallas_call(
        _h0_body,
        grid=(NBLK,),
        in_specs=[
            pl.BlockSpec((BN, D), lambda i: (i, 0)),
            pl.BlockSpec((BN, PE), lambda i: (i, 0)),
            pl.BlockSpec((D, D), lambda i: (0, 0)),
            pl.BlockSpec((PE, D), lambda i: (0, 0)),
        ],
        out_specs=pl.BlockSpec((BN, D), lambda i: (i, 0)),
        out_shape=jax.ShapeDtypeStruct((N, D), jnp.float32),
    )(x, RWPE, W_in, W_pe)


def _sc_body(h0_hbm, src_hbm, dst_hbm, zrow_hbm, ones_hbm, agg_hbm, deg_hbm,
             sidx0, sidx1, didx, rows0, rows1, onesb, acc,
             sem0, sem1, ssem0, ssem1):
    cid = lax.axis_index("c")
    sid = lax.axis_index("s")
    wid = sid * SC_CORES + cid
    r0 = sid * RPT
    pltpu.sync_copy(dst_hbm.at[wid], didx)
    pltpu.sync_copy(ones_hbm, onesb)
    pltpu.sync_copy(zrow_hbm, acc.at[pl.ds(r0, RPT)])
    plsc.subcore_barrier()

    def dchunk(k, carry):
        pltpu.sync_copy(onesb, acc.at[didx.at[k]], add=True)
        return carry

    lax.fori_loop(0, NCHUNK, dchunk, 0)
    plsc.subcore_barrier()
    pltpu.sync_copy(acc.at[pl.ds(r0, RPT)], deg_hbm.at[cid, pl.ds(r0, RPT)])
    pltpu.sync_copy(zrow_hbm, acc.at[pl.ds(r0, RPT)])
    plsc.subcore_barrier()
    ebase = wid * EPW
    bufs = ((sidx0, rows0, sem0, ssem0), (sidx1, rows1, sem1, ssem1))

    def src_slice(k):
        kk = jnp.minimum(k, NCHUNK - 1)
        return src_hbm.at[pl.ds(ebase + kk * CB, CB)]

    pltpu.sync_copy(src_slice(0), sidx0)
    pltpu.async_copy(h0_hbm.at[sidx0], rows0, sem0)
    pltpu.async_copy(src_slice(1), sidx1, ssem1)

    def step(k, cur, nxt):
        pltpu.make_async_copy(h0_hbm.at[cur[0]], cur[1], cur[2]).wait()
        pltpu.async_copy(src_slice(k + 2), cur[0], cur[3])
        pltpu.make_async_copy(src_slice(k + 1), nxt[0], nxt[3]).wait()
        pltpu.async_copy(h0_hbm.at[nxt[0]], nxt[1], nxt[2])
        pltpu.sync_copy(cur[1], acc.at[didx.at[k]], add=True)

    def pair(j, carry):
        k = 2 * j
        step(k, bufs[0], bufs[1])
        step(k + 1, bufs[1], bufs[0])
        return carry

    lax.fori_loop(0, (NCHUNK - 1) // 2, pair, 0)
    pltpu.make_async_copy(h0_hbm.at[bufs[0][0]], bufs[0][1], bufs[0][2]).wait()
    pltpu.sync_copy(bufs[0][1], acc.at[didx.at[NCHUNK - 1]], add=True)
    pltpu.make_async_copy(src_slice(NCHUNK - 1), bufs[1][0], bufs[1][3]).wait()
    plsc.subcore_barrier()
    pltpu.sync_copy(acc.at[pl.ds(r0, RPT)], agg_hbm.at[cid, pl.ds(r0, RPT)])


def _sc_mesh():
    return plsc.VectorSubcoreMesh(core_axis_name="c", subcore_axis_name="s",
                                  num_cores=SC_CORES, num_subcores=SC_SUBCORES)


@functools.cache
def _sc_kernel():
    return functools.partial(
        pl.kernel,
        out_type=(
            jax.ShapeDtypeStruct((SC_CORES, N_PAD, D), jnp.float32),
            jax.ShapeDtypeStruct((SC_CORES, N_PAD, D), jnp.float32),
        ),
        mesh=_sc_mesh(),
        scratch_types=(
            pltpu.VMEM((CB,), jnp.int32),
            pltpu.VMEM((CB,), jnp.int32),
            pltpu.VMEM((NCHUNK, CB), jnp.int32),
            pltpu.VMEM((CB, D), jnp.float32),
            pltpu.VMEM((CB, D), jnp.float32),
            pltpu.VMEM((CB, D), jnp.float32),
            pltpu.VMEM_SHARED((N_PAD, D), jnp.float32),
            pltpu.SemaphoreType.DMA,
            pltpu.SemaphoreType.DMA,
            pltpu.SemaphoreType.DMA,
            pltpu.SemaphoreType.DMA,
        ),
    )(_sc_body)


def _sc_agg_deg(h0, src, dst, zrow, onesh):
    return _sc_kernel()(h0, src, dst.reshape(NW, NCHUNK, CB), zrow, onesh)


def _pool_route_body(aggp_ref, degp_ref, h0_ref, wa_ref, ws_ref, b_ref,
                     ew1, eb1, ew2, eb2, ptT, pt, br, dw1, db1, dw2, db2, ap,
                     lp_ref, ti_ref, ps_ref, cnt_ref):
    i = pl.program_id(0)
    deg = jnp.maximum(degp_ref[0] + degp_ref[1], 1.0)
    agg = (aggp_ref[0] + aggp_ref[1]) / deg
    h = jnp.maximum(_dot(agg, wa_ref[...]) + _dot(h0_ref[...], ws_ref[...]), 0.0)
    seg = b_ref[...]
    gid = lax.broadcasted_iota(jnp.int32, (BN, G), 1).astype(jnp.float32)
    oh = (seg == gid).astype(jnp.float32)
    dn = (((0,), (0,)), ((), ()))
    hp = lax.Precision.HIGHEST
    psum = lax.dot_general(oh, h, dn,
                           preferred_element_type=jnp.float32, precision=hp)
    csum = lax.dot_general(oh, jnp.ones((BN, D), jnp.float32), dn,
                           preferred_element_type=jnp.float32, precision=hp)

    @pl.when(i == 0)
    def _():
        ps_ref[...] = jnp.zeros_like(ps_ref)
        cnt_ref[...] = jnp.zeros_like(cnt_ref)

    ps_ref[...] += psum
    cnt_ref[...] += csum

    @pl.when(i == NBLK - 1)
    def _():
        pooled = ps_ref[...] / jnp.maximum(cnt_ref[...], 1.0)
        t = jnp.maximum(_dot(pooled, ew1[...]) + eb1[...], 0.0)
        z = _dot(t, ew2[...]) + eb2[...]

        iota_p = lax.broadcasted_iota(jnp.int32, (G, P), 1)
        col32 = lax.broadcasted_iota(jnp.int32, (G, H * K1), 1)
        scale = np.float32(1.0 / np.sqrt(DH))
        ti = jnp.zeros((G, H * K1), jnp.int32)
        c_parts = []
        for hh in range(H):
            zh = z[:, hh * DH:(hh + 1) * DH]
            lg = _dot(zh, ptT[hh]) * scale
            work = lg + br[...]
            sels, ohs, idxs = [], [], []
            for j in range(K1):
                m = jnp.max(work, axis=1, keepdims=True)
                idx = jnp.min(jnp.where(work == m, iota_p, P), axis=1,
                              keepdims=True)
                ohj = iota_p == idx
                sel = jnp.sum(jnp.where(ohj, lg, 0.0), axis=1, keepdims=True)
                work = jnp.where(ohj, np.float32(-3.0e38), work)
                sels.append(sel)
                ohs.append(ohj)
                idxs.append(idx)
            m8 = sels[0]
            for j in range(1, K1):
                m8 = jnp.maximum(m8, sels[j])
            es = [jnp.exp(x - m8) for x in sels]
            tot = es[0]
            for j in range(1, K1):
                tot = tot + es[j]
            rt = 1.0 / tot
            wcomb = jnp.zeros((G, P), jnp.float32)
            for j in range(K1):
                wcomb = wcomb + jnp.where(ohs[j], es[j] * rt, 0.0)
            c_parts.append(_dotx(wcomb, pt[hh]))
            for j in range(K1):
                ti = jnp.where(col32 == (hh * K1 + j), idxs[j], ti)

        c = jnp.concatenate(c_parts, axis=1)
        sg = 1.0 / (1.0 + jnp.exp(-ap[0, 0]))
        o = sg * c
        t2 = jnp.maximum(_dot(o, dw1[...]) + db1[...], 0.0)
        lgts = _dot(t2, dw2[...]) + db2[...]
        colD = lax.broadcasted_iota(jnp.int32, (G, 128), 1)
        valid = colD < NC_OUT
        mx = jnp.max(jnp.where(valid, lgts, np.float32(-3.0e38)), axis=1,
                     keepdims=True)
        ex = jnp.where(valid, jnp.exp(lgts - mx), 0.0)
        lse = jnp.log(jnp.sum(ex, axis=1, keepdims=True))
        lp = lgts - mx - lse
        lp_ref[...] = lax.slice(lp, (0, 0), (G, NC_OUT))
        ti_ref[...] = ti


def _pool_route(aggp, degp, h0, W_agg, W_self, batchf, enc_W1, eb1, enc_W2,
                eb2, protoT, protos, biasr, dec_W1, db1, dW2p, db2p, ap):
    full = lambda shape: pl.BlockSpec(shape, lambda i: tuple(0 for _ in shape))
    return pl.pallas_call(
        _pool_route_body,
        grid=(NBLK,),
        in_specs=[
            pl.BlockSpec((SC_CORES, BN, D), lambda i: (0, i, 0)),
            pl.BlockSpec((SC_CORES, BN, D), lambda i: (0, i, 0)),
            pl.BlockSpec((BN, D), lambda i: (i, 0)),
            full((D, D)),
            full((D, D)),
            pl.BlockSpec((BN, 1), lambda i: (i, 0)),
            full((D, HID)),
            full((1, HID)),
            full((HID, D)),
            full((1, D)),
            full((H, DH, P)),
            full((H, P, DH)),
            full((1, P)),
            full((D, HID)),
            full((1, HID)),
            full((HID, 128)),
            full((1, 128)),
            full((1, 1)),
        ],
        out_specs=[
            full((G, NC_OUT)),
            full((G, H * K1)),
        ],
        out_shape=[
            jax.ShapeDtypeStruct((G, NC_OUT), jnp.float32),
            jax.ShapeDtypeStruct((G, H * K1), jnp.int32),
        ],
        scratch_shapes=[
            pltpu.VMEM((G, D), jnp.float32),
            pltpu.VMEM((G, D), jnp.float32),
        ],
    )(aggp, degp, h0, W_agg, W_self, batchf, enc_W1, eb1, enc_W2, eb2,
      protoT, protos, biasr, dec_W1, db1, dW2p, db2p, ap)


def kernel(x, RWPE, adj_t, batch, index, bias, alpha,
           W_in, W_pe, W_self, W_agg,
           enc_W1, enc_b1, enc_W2, enc_b2,
           prototypes, dec_W1, dec_b1, dec_W2, dec_b2, alpha_p):
    src = adj_t[0]
    dst = adj_t[1]

    h0 = _compute_h0(x, RWPE, W_in, W_pe)

    zrow = jnp.zeros((RPT, D), jnp.float32)
    onesh = jnp.ones((CB, D), jnp.float32)
    aggp, degp = _sc_agg_deg(h0, src, dst, zrow, onesh)

    batchf = batch.astype(jnp.float32).reshape(N, 1)
    protoT = jnp.transpose(prototypes, (0, 2, 1))
    lp, ti = _pool_route(
        aggp, degp, h0, W_agg, W_self, batchf,
        enc_W1, enc_b1.reshape(1, HID), enc_W2, enc_b2.reshape(1, D),
        protoT, prototypes, bias.reshape(1, P),
        dec_W1, dec_b1.reshape(1, HID),
        jnp.pad(dec_W2, ((0, 0), (0, 128 - NC_OUT))),
        jnp.pad(dec_b2, (0, 128 - NC_OUT)).reshape(1, 128),
        jnp.asarray(alpha_p, jnp.float32).reshape(1, 1),
    )
    return (lp, ti.reshape(G, H, K1))
